# baseline (speedup 1.0000x reference)
import functools

import jax
import jax.numpy as jnp
from jax import lax
from jax.experimental import pallas as pl
from jax.experimental.pallas import tpu as pltpu
from jax.experimental.pallas import tpu_sc as plsc

_N = 1024
_D = 64
_K = 1000
_KP = 1024


def _sort_kernel(ct_ref, vs_ref, ms_ref):
    v = ct_ref[...]
    i = jax.lax.broadcasted_iota(jnp.int32, (_KP, _D), 0)
    row = jax.lax.broadcasted_iota(jnp.int32, (_KP, _D), 0)

    for kk in range(1, 11):
        blk = 1 << kk
        desc = (row & blk) != 0
        for jj in range(kk - 1, -1, -1):
            s = 1 << jj
            upper = (row & s) != 0
            flip = jnp.logical_xor(upper, desc)
            pv = jnp.where(upper, jnp.roll(v, s, axis=0), jnp.roll(v, -s, axis=0))
            pi = jnp.where(upper, jnp.roll(i, s, axis=0), jnp.roll(i, -s, axis=0))
            lt = (pv < v) | ((pv == v) & (pi < i))
            take = jnp.logical_xor(lt, flip)
            v = jnp.where(take, pv, v)
            i = jnp.where(take, pi, i)

    m = i
    for dd in [1, 2, 4, 8, 16, 32, 64, 128, 256, 512]:
        ok = (row >= dd) & (v == jnp.roll(v, dd, axis=0))
        m = jnp.where(ok, jnp.minimum(m, jnp.roll(m, dd, axis=0)), m)

    vs_ref[...] = v
    ms_ref[...] = m


def _make_search_kernel():
    mesh = plsc.VectorSubcoreMesh(core_axis_name="c", subcore_axis_name="s")

    @functools.partial(
        pl.kernel,
        mesh=mesh,
        out_type=jax.ShapeDtypeStruct((_D, _KP), jnp.int32),
        compiler_params=pltpu.CompilerParams(needs_layout_passes=False),
        scratch_types=[
            pltpu.VMEM((_KP,), jnp.float32),
            pltpu.VMEM((_KP,), jnp.int32),
            pltpu.VMEM((_KP,), jnp.float32),
            pltpu.VMEM((_KP,), jnp.int32),
        ],
    )
    def search_kernel(vs_hbm, ms_hbm, ft_hbm, mi_hbm, vrow, mrow, frow, omin):
        wid = lax.axis_index("s") * 2 + lax.axis_index("c")
        for rj in range(2):
            j = wid * 2 + rj
            pltpu.sync_copy(vs_hbm.at[j], vrow)
            pltpu.sync_copy(ms_hbm.at[j], mrow)
            pltpu.sync_copy(ft_hbm.at[j], frow)

            _U = 4

            def qbody(qi, carry):
                xs = [frow[pl.ds((qi * _U + u) * 16, 16)] for u in range(_U)]
                rs = [jnp.zeros((16,), jnp.int32) for _ in range(_U)]
                for step in (512, 256, 128, 64, 32, 16, 8, 4, 2, 1):
                    vals = []
                    for u in range(_U):
                        cand = rs[u] + step
                        vals.append((cand, plsc.load_gather(vrow, [cand - 1])))
                    for u in range(_U):
                        cand, val = vals[u]
                        rs[u] = jnp.where(val < xs[u], cand, rs[u])
                for u in range(_U):
                    x = xs[u]
                    r = rs[u]
                    l = jnp.maximum(r - 1, 0)
                    rr = jnp.minimum(r, _KP - 1)
                    vl = plsc.load_gather(vrow, [l])
                    vr = plsc.load_gather(vrow, [rr])
                    ml_ = plsc.load_gather(mrow, [l])
                    mr_ = plsc.load_gather(mrow, [rr])
                    dl = x - vl
                    dr = x - vr
                    sql = dl * dl
                    sqr = dr * dr
                    big = jnp.float32(jnp.inf)
                    sql = jnp.where(r >= 1, sql, big)
                    sqr = jnp.where(r <= _K - 1, sqr, big)
                    mini = jnp.where(sql < sqr, ml_,
                                     jnp.where(sqr < sql, mr_, jnp.minimum(ml_, mr_)))
                    omin[pl.ds((qi * _U + u) * 16, 16)] = mini
                return carry

            jax.lax.fori_loop(0, _N // 16 // _U, qbody, 0)
            pltpu.sync_copy(omin, mi_hbm.at[j])

    return search_kernel


def _loss_kernel(f_ref, cpad_ref, fT_ref, extv_ref, exti_ref, miT_ref, out_ref):
    f = f_ref[...]

    fT = fT_ref[...]
    lo_vb = jnp.broadcast_to(extv_ref[:, 0:1], (_D, _N))
    hi_vb = jnp.broadcast_to(extv_ref[:, 1:2], (_D, _N))
    lo_ib = jnp.broadcast_to(exti_ref[:, 0:1], (_D, _N))
    hi_ib = jnp.broadcast_to(exti_ref[:, 1:2], (_D, _N))
    dl = fT - lo_vb
    dh = fT - hi_vb
    sql = dl * dl
    sqh = dh * dh
    maT = jnp.where(sql > sqh, lo_ib,
                    jnp.where(sqh > sql, hi_ib, jnp.minimum(lo_ib, hi_ib)))

    def mode(idxT):
        counts = jnp.zeros((_D, _N), jnp.int32)
        for r in range(_D):
            rolled = jnp.roll(idxT, r, axis=0) if r else idxT
            counts = counts + (rolled == idxT).astype(jnp.int32)
        key = counts * 1024 + (1023 - idxT)
        mkey = jnp.max(key, axis=0, keepdims=True)
        mode_row = 1023 - jnp.bitwise_and(mkey, 1023)
        return mode_row.reshape(_N, 1)

    mode_min = mode(miT_ref[...])
    mode_max = mode(maT)

    iota_m = jax.lax.broadcasted_iota(jnp.int32, (_N, _N), 1)
    oh_p = (mode_min == iota_m).astype(jnp.float32)
    oh_n = (mode_max == iota_m).astype(jnp.float32)
    cpad = cpad_ref[...]
    pos = jnp.dot(oh_p, cpad, preferred_element_type=jnp.float32)
    neg = jnp.dot(oh_n, cpad, preferred_element_type=jnp.float32)

    eps = jnp.float32(1e-6)

    def pdist(a, b):
        d = a - b + eps
        return jnp.sqrt(jnp.sum(d * d, axis=1, keepdims=True))

    d_ap = pdist(f, pos)
    d_an = pdist(f, neg)
    d_pn = pdist(pos, neg)
    d_neg = jnp.minimum(d_an, d_pn)
    li = jnp.maximum(d_ap - d_neg + 1.0, 0.0)
    loss = jnp.sum(li) * jnp.float32(1.0 / _N)
    out_ref[...] = jnp.full((8, 128), loss, jnp.float32)


def kernel(input_features, centroids):
    f = input_features.astype(jnp.float32)
    c = centroids.astype(jnp.float32)

    inf = jnp.float32(jnp.inf)
    cpadinf = jnp.concatenate(
        [c, jnp.full((_KP - _K, _D), inf, jnp.float32)], axis=0)
    fT = f.T

    vsort_c, msort_c = pl.pallas_call(
        _sort_kernel,
        out_shape=[jax.ShapeDtypeStruct((_KP, _D), jnp.float32),
                   jax.ShapeDtypeStruct((_KP, _D), jnp.int32)],
    )(cpadinf)
    vsort = vsort_c.T
    msort = msort_c.T

    miT = _make_search_kernel()(vsort, msort, fT)

    zpadv = jnp.zeros((_D, 6), jnp.float32)
    zpadi = jnp.zeros((_D, 6), jnp.int32)
    extv = jnp.concatenate(
        [vsort_c[0:1, :].T, vsort_c[_K - 1:_K, :].T, zpadv], axis=1)
    exti = jnp.concatenate(
        [msort_c[0:1, :].T, msort_c[_K - 1:_K, :].T, zpadi], axis=1)

    cpad = jnp.concatenate([c, jnp.zeros((_N - _K, _D), jnp.float32)], axis=0)
    out = pl.pallas_call(
        _loss_kernel,
        out_shape=jax.ShapeDtypeStruct((8, 128), jnp.float32),
    )(f, cpad, fT, extv, exti, miT)
    return out[0, 0]


# speedup vs baseline: 1.1315x; 1.1315x over previous
import functools

import jax
import jax.numpy as jnp
from jax import lax
from jax.experimental import pallas as pl
from jax.experimental.pallas import tpu as pltpu
from jax.experimental.pallas import tpu_sc as plsc

_N = 1024
_D = 64
_K = 1000
_KP = 1024


def _sort_kernel(ct_ref, vs_ref, ms_ref):
    v = ct_ref[...]
    i = jax.lax.broadcasted_iota(jnp.int32, (_D, _KP), 1)
    lane = jax.lax.broadcasted_iota(jnp.int32, (_D, _KP), 1)

    for kk in range(1, 11):
        blk = 1 << kk
        desc = (lane & blk) != 0
        for jj in range(kk - 1, -1, -1):
            s = 1 << jj
            upper = (lane & s) != 0
            flip = jnp.logical_xor(upper, desc)
            pv = jnp.where(upper, jnp.roll(v, s, axis=1), jnp.roll(v, -s, axis=1))
            pi = jnp.where(upper, jnp.roll(i, s, axis=1), jnp.roll(i, -s, axis=1))
            lt = (pv < v) | ((pv == v) & (pi < i))
            take = jnp.logical_xor(lt, flip)
            v = jnp.where(take, pv, v)
            i = jnp.where(take, pi, i)

    m = i
    for dd in [1, 2, 4, 8, 16, 32, 64, 128, 256, 512]:
        ok = (lane >= dd) & (v == jnp.roll(v, dd, axis=1))
        m = jnp.where(ok, jnp.minimum(m, jnp.roll(m, dd, axis=1)), m)

    vs_ref[...] = v
    ms_ref[...] = m


def _make_search_kernel():
    mesh = plsc.VectorSubcoreMesh(core_axis_name="c", subcore_axis_name="s")

    @functools.partial(
        pl.kernel,
        mesh=mesh,
        out_type=jax.ShapeDtypeStruct((_D, _KP), jnp.int32),
        compiler_params=pltpu.CompilerParams(needs_layout_passes=False),
        scratch_types=[
            pltpu.VMEM((_KP,), jnp.float32),
            pltpu.VMEM((_KP,), jnp.int32),
            pltpu.VMEM((_KP,), jnp.float32),
            pltpu.VMEM((_KP,), jnp.int32),
        ],
    )
    def search_kernel(vs_hbm, ms_hbm, ft_hbm, mi_hbm, vrow, mrow, frow, omin):
        wid = lax.axis_index("s") * 2 + lax.axis_index("c")
        for rj in range(2):
            j = wid * 2 + rj
            pltpu.sync_copy(vs_hbm.at[j], vrow)
            pltpu.sync_copy(ms_hbm.at[j], mrow)
            pltpu.sync_copy(ft_hbm.at[j], frow)

            _U = 4

            def qbody(qi, carry):
                xs = [frow[pl.ds((qi * _U + u) * 16, 16)] for u in range(_U)]
                rs = [jnp.zeros((16,), jnp.int32) for _ in range(_U)]
                for step in (512, 256, 128, 64, 32, 16, 8, 4, 2, 1):
                    vals = []
                    for u in range(_U):
                        cand = rs[u] + step
                        vals.append((cand, plsc.load_gather(vrow, [cand - 1])))
                    for u in range(_U):
                        cand, val = vals[u]
                        rs[u] = jnp.where(val < xs[u], cand, rs[u])
                for u in range(_U):
                    x = xs[u]
                    r = rs[u]
                    l = jnp.maximum(r - 1, 0)
                    rr = jnp.minimum(r, _KP - 1)
                    vl = plsc.load_gather(vrow, [l])
                    vr = plsc.load_gather(vrow, [rr])
                    ml_ = plsc.load_gather(mrow, [l])
                    mr_ = plsc.load_gather(mrow, [rr])
                    dl = x - vl
                    dr = x - vr
                    sql = dl * dl
                    sqr = dr * dr
                    big = jnp.float32(jnp.inf)
                    sql = jnp.where(r >= 1, sql, big)
                    sqr = jnp.where(r <= _K - 1, sqr, big)
                    mini = jnp.where(sql < sqr, ml_,
                                     jnp.where(sqr < sql, mr_, jnp.minimum(ml_, mr_)))
                    omin[pl.ds((qi * _U + u) * 16, 16)] = mini
                return carry

            jax.lax.fori_loop(0, _N // 16 // _U, qbody, 0)
            pltpu.sync_copy(omin, mi_hbm.at[j])

    return search_kernel


def _loss_kernel(f_ref, cpad_ref, fT_ref, extv_ref, exti_ref, miT_ref, out_ref):
    f = f_ref[...]

    fT = fT_ref[...]
    lo_vb = jnp.broadcast_to(extv_ref[:, 0:1], (_D, _N))
    hi_vb = jnp.broadcast_to(extv_ref[:, 1:2], (_D, _N))
    lo_ib = jnp.broadcast_to(exti_ref[:, 0:1], (_D, _N))
    hi_ib = jnp.broadcast_to(exti_ref[:, 1:2], (_D, _N))
    dl = fT - lo_vb
    dh = fT - hi_vb
    sql = dl * dl
    sqh = dh * dh
    maT = jnp.where(sql > sqh, lo_ib,
                    jnp.where(sqh > sql, hi_ib, jnp.minimum(lo_ib, hi_ib)))

    def mode(idxT):
        counts = jnp.zeros((_D, _N), jnp.int32)
        for r in range(_D):
            rolled = jnp.roll(idxT, r, axis=0) if r else idxT
            counts = counts + (rolled == idxT).astype(jnp.int32)
        key = counts * 1024 + (1023 - idxT)
        mkey = jnp.max(key, axis=0, keepdims=True)
        mode_row = 1023 - jnp.bitwise_and(mkey, 1023)
        return mode_row.reshape(_N, 1)

    mode_min = mode(miT_ref[...])
    mode_max = mode(maT)

    iota_m = jax.lax.broadcasted_iota(jnp.int32, (_N, _N), 1)
    oh_p = (mode_min == iota_m).astype(jnp.float32)
    oh_n = (mode_max == iota_m).astype(jnp.float32)
    cpad = cpad_ref[...]
    pos = jnp.dot(oh_p, cpad, preferred_element_type=jnp.float32)
    neg = jnp.dot(oh_n, cpad, preferred_element_type=jnp.float32)

    eps = jnp.float32(1e-6)

    def pdist(a, b):
        d = a - b + eps
        return jnp.sqrt(jnp.sum(d * d, axis=1, keepdims=True))

    d_ap = pdist(f, pos)
    d_an = pdist(f, neg)
    d_pn = pdist(pos, neg)
    d_neg = jnp.minimum(d_an, d_pn)
    li = jnp.maximum(d_ap - d_neg + 1.0, 0.0)
    loss = jnp.sum(li) * jnp.float32(1.0 / _N)
    out_ref[...] = jnp.full((8, 128), loss, jnp.float32)


def kernel(input_features, centroids):
    f = input_features.astype(jnp.float32)
    c = centroids.astype(jnp.float32)

    inf = jnp.float32(jnp.inf)
    cT = jnp.concatenate(
        [c, jnp.full((_KP - _K, _D), inf, jnp.float32)], axis=0).T
    fT = f.T

    vsort, msort = pl.pallas_call(
        _sort_kernel,
        out_shape=[jax.ShapeDtypeStruct((_D, _KP), jnp.float32),
                   jax.ShapeDtypeStruct((_D, _KP), jnp.int32)],
    )(cT)

    miT = _make_search_kernel()(vsort, msort, fT)

    zpadv = jnp.zeros((_D, 6), jnp.float32)
    zpadi = jnp.zeros((_D, 6), jnp.int32)
    extv = jnp.concatenate([vsort[:, 0:1], vsort[:, _K - 1:_K], zpadv], axis=1)
    exti = jnp.concatenate([msort[:, 0:1], msort[:, _K - 1:_K], zpadi], axis=1)

    cpad = jnp.concatenate([c, jnp.zeros((_N - _K, _D), jnp.float32)], axis=0)
    out = pl.pallas_call(
        _loss_kernel,
        out_shape=jax.ShapeDtypeStruct((8, 128), jnp.float32),
    )(f, cpad, fT, extv, exti, miT)
    return out[0, 0]


# speedup vs baseline: 1.1475x; 1.0141x over previous
import functools

import jax
import jax.numpy as jnp
from jax import lax
from jax.experimental import pallas as pl
from jax.experimental.pallas import tpu as pltpu
from jax.experimental.pallas import tpu_sc as plsc

_N = 1024
_D = 64
_K = 1000
_KP = 1024


def _sort_kernel(ct_ref, vs_ref, ms_ref):
    v = ct_ref[...]
    i = jax.lax.broadcasted_iota(jnp.int32, (_D, _KP), 1)
    lane = jax.lax.broadcasted_iota(jnp.int32, (_D, _KP), 1)

    for kk in range(1, 11):
        blk = 1 << kk
        desc = (lane & blk) != 0
        for jj in range(kk - 1, -1, -1):
            s = 1 << jj
            upper = (lane & s) != 0
            flip = jnp.logical_xor(upper, desc)
            pv = jnp.where(upper, jnp.roll(v, s, axis=1), jnp.roll(v, -s, axis=1))
            pi = jnp.where(upper, jnp.roll(i, s, axis=1), jnp.roll(i, -s, axis=1))
            lt = (pv < v) | ((pv == v) & (pi < i))
            take = jnp.logical_xor(lt, flip)
            v = jnp.where(take, pv, v)
            i = jnp.where(take, pi, i)

    m = i
    for dd in [1, 2, 4, 8, 16, 32, 64, 128, 256, 512]:
        ok = (lane >= dd) & (v == jnp.roll(v, dd, axis=1))
        m = jnp.where(ok, jnp.minimum(m, jnp.roll(m, dd, axis=1)), m)

    vs_ref[...] = v
    ms_ref[...] = m


def _make_search_kernel():
    mesh = plsc.VectorSubcoreMesh(core_axis_name="c", subcore_axis_name="s")

    @functools.partial(
        pl.kernel,
        mesh=mesh,
        out_type=jax.ShapeDtypeStruct((_D, _KP), jnp.int32),
        compiler_params=pltpu.CompilerParams(needs_layout_passes=False),
        scratch_types=[
            pltpu.VMEM((_KP,), jnp.float32),
            pltpu.VMEM((_KP,), jnp.int32),
            pltpu.VMEM((_KP,), jnp.float32),
            pltpu.VMEM((_KP,), jnp.int32),
        ],
    )
    def search_kernel(vs_hbm, ms_hbm, ft_hbm, mi_hbm, vrow, mrow, frow, omin):
        wid = lax.axis_index("s") * 2 + lax.axis_index("c")
        for rj in range(2):
            j = wid * 2 + rj
            pltpu.sync_copy(vs_hbm.at[j], vrow)
            pltpu.sync_copy(ms_hbm.at[j], mrow)
            pltpu.sync_copy(ft_hbm.at[j], frow)

            _U = 8

            def qbody(qi, carry):
                xs = [frow[pl.ds((qi * _U + u) * 16, 16)] for u in range(_U)]
                rs = [jnp.zeros((16,), jnp.int32) for _ in range(_U)]
                for step in (512, 256, 128, 64, 32, 16, 8, 4, 2, 1):
                    vals = []
                    for u in range(_U):
                        cand = rs[u] + step
                        vals.append((cand, plsc.load_gather(vrow, [cand - 1])))
                    for u in range(_U):
                        cand, val = vals[u]
                        rs[u] = jnp.where(val < xs[u], cand, rs[u])
                for u in range(_U):
                    x = xs[u]
                    r = rs[u]
                    l = jnp.maximum(r - 1, 0)
                    rr = jnp.minimum(r, _KP - 1)
                    vl = plsc.load_gather(vrow, [l])
                    vr = plsc.load_gather(vrow, [rr])
                    ml_ = plsc.load_gather(mrow, [l])
                    mr_ = plsc.load_gather(mrow, [rr])
                    dl = x - vl
                    dr = x - vr
                    sql = dl * dl
                    sqr = dr * dr
                    big = jnp.float32(jnp.inf)
                    sql = jnp.where(r >= 1, sql, big)
                    sqr = jnp.where(r <= _K - 1, sqr, big)
                    mini = jnp.where(sql < sqr, ml_,
                                     jnp.where(sqr < sql, mr_, jnp.minimum(ml_, mr_)))
                    omin[pl.ds((qi * _U + u) * 16, 16)] = mini
                return carry

            jax.lax.fori_loop(0, _N // 16 // _U, qbody, 0)
            pltpu.sync_copy(omin, mi_hbm.at[j])

    return search_kernel


def _loss_kernel(f_ref, cpad_ref, fT_ref, extv_ref, exti_ref, miT_ref, out_ref):
    f = f_ref[...]

    fT = fT_ref[...]
    lo_vb = jnp.broadcast_to(extv_ref[:, 0:1], (_D, _N))
    hi_vb = jnp.broadcast_to(extv_ref[:, 1:2], (_D, _N))
    lo_ib = jnp.broadcast_to(exti_ref[:, 0:1], (_D, _N))
    hi_ib = jnp.broadcast_to(exti_ref[:, 1:2], (_D, _N))
    dl = fT - lo_vb
    dh = fT - hi_vb
    sql = dl * dl
    sqh = dh * dh
    maT = jnp.where(sql > sqh, lo_ib,
                    jnp.where(sqh > sql, hi_ib, jnp.minimum(lo_ib, hi_ib)))

    def mode(idxT):
        counts = jnp.zeros((_D, _N), jnp.int32)
        for r in range(_D):
            rolled = jnp.roll(idxT, r, axis=0) if r else idxT
            counts = counts + (rolled == idxT).astype(jnp.int32)
        key = counts * 1024 + (1023 - idxT)
        mkey = jnp.max(key, axis=0, keepdims=True)
        mode_row = 1023 - jnp.bitwise_and(mkey, 1023)
        return mode_row.reshape(_N, 1)

    mode_min = mode(miT_ref[...])
    mode_max = mode(maT)

    iota_m = jax.lax.broadcasted_iota(jnp.int32, (_N, _N), 1)
    oh_p = (mode_min == iota_m).astype(jnp.float32)
    oh_n = (mode_max == iota_m).astype(jnp.float32)
    cpad = cpad_ref[...]
    pos = jnp.dot(oh_p, cpad, preferred_element_type=jnp.float32)
    neg = jnp.dot(oh_n, cpad, preferred_element_type=jnp.float32)

    eps = jnp.float32(1e-6)

    def pdist(a, b):
        d = a - b + eps
        return jnp.sqrt(jnp.sum(d * d, axis=1, keepdims=True))

    d_ap = pdist(f, pos)
    d_an = pdist(f, neg)
    d_pn = pdist(pos, neg)
    d_neg = jnp.minimum(d_an, d_pn)
    li = jnp.maximum(d_ap - d_neg + 1.0, 0.0)
    loss = jnp.sum(li) * jnp.float32(1.0 / _N)
    out_ref[...] = jnp.full((8, 128), loss, jnp.float32)


def kernel(input_features, centroids):
    f = input_features.astype(jnp.float32)
    c = centroids.astype(jnp.float32)

    inf = jnp.float32(jnp.inf)
    cT = jnp.concatenate(
        [c, jnp.full((_KP - _K, _D), inf, jnp.float32)], axis=0).T
    fT = f.T

    vsort, msort = pl.pallas_call(
        _sort_kernel,
        out_shape=[jax.ShapeDtypeStruct((_D, _KP), jnp.float32),
                   jax.ShapeDtypeStruct((_D, _KP), jnp.int32)],
    )(cT)

    miT = _make_search_kernel()(vsort, msort, fT)

    zpadv = jnp.zeros((_D, 6), jnp.float32)
    zpadi = jnp.zeros((_D, 6), jnp.int32)
    extv = jnp.concatenate([vsort[:, 0:1], vsort[:, _K - 1:_K], zpadv], axis=1)
    exti = jnp.concatenate([msort[:, 0:1], msort[:, _K - 1:_K], zpadi], axis=1)

    cpad = jnp.concatenate([c, jnp.zeros((_N - _K, _D), jnp.float32)], axis=0)
    out = pl.pallas_call(
        _loss_kernel,
        out_shape=jax.ShapeDtypeStruct((8, 128), jnp.float32),
    )(f, cpad, fT, extv, exti, miT)
    return out[0, 0]


# speedup vs baseline: 1.2425x; 1.0828x over previous
import functools

import jax
import jax.numpy as jnp
from jax import lax
from jax.experimental import pallas as pl
from jax.experimental.pallas import tpu as pltpu
from jax.experimental.pallas import tpu_sc as plsc

_N = 1024
_D = 64
_K = 1000
_KP = 1024


def _sort_kernel(ct_ref, vs_ref, ms_ref):
    v = ct_ref[...]
    i = jax.lax.broadcasted_iota(jnp.int32, (_D, _KP), 1)
    lane = jax.lax.broadcasted_iota(jnp.int32, (_D, _KP), 1)

    for kk in range(1, 11):
        blk = 1 << kk
        desc = (lane & blk) != 0
        for jj in range(kk - 1, -1, -1):
            s = 1 << jj
            upper = (lane & s) != 0
            flip = jnp.logical_xor(upper, desc)
            pv = jnp.where(upper, jnp.roll(v, s, axis=1), jnp.roll(v, -s, axis=1))
            pi = jnp.where(upper, jnp.roll(i, s, axis=1), jnp.roll(i, -s, axis=1))
            lt = (pv < v) | ((pv == v) & upper)
            take = jnp.logical_xor(lt, flip)
            v = jnp.where(take, pv, v)
            i = jnp.where(take, pi, i)

    m = i
    for dd in [1, 2, 4, 8, 16, 32, 64, 128, 256, 512]:
        ok = (lane >= dd) & (v == jnp.roll(v, dd, axis=1))
        m = jnp.where(ok, jnp.minimum(m, jnp.roll(m, dd, axis=1)), m)

    vs_ref[...] = v
    ms_ref[...] = m


def _make_search_kernel():
    mesh = plsc.VectorSubcoreMesh(core_axis_name="c", subcore_axis_name="s")

    @functools.partial(
        pl.kernel,
        mesh=mesh,
        out_type=jax.ShapeDtypeStruct((_D, _KP), jnp.int32),
        compiler_params=pltpu.CompilerParams(needs_layout_passes=False),
        scratch_types=[
            pltpu.VMEM((_KP,), jnp.float32),
            pltpu.VMEM((_KP,), jnp.int32),
            pltpu.VMEM((_KP,), jnp.float32),
            pltpu.VMEM((_KP,), jnp.int32),
        ],
    )
    def search_kernel(vs_hbm, ms_hbm, ft_hbm, mi_hbm, vrow, mrow, frow, omin):
        wid = lax.axis_index("s") * 2 + lax.axis_index("c")
        for rj in range(2):
            j = wid * 2 + rj
            pltpu.sync_copy(vs_hbm.at[j], vrow)
            pltpu.sync_copy(ms_hbm.at[j], mrow)
            pltpu.sync_copy(ft_hbm.at[j], frow)

            _U = 8

            def qbody(qi, carry):
                xs = [frow[pl.ds((qi * _U + u) * 16, 16)] for u in range(_U)]
                rs = [jnp.zeros((16,), jnp.int32) for _ in range(_U)]
                for step in (512, 256, 128, 64, 32, 16, 8, 4, 2, 1):
                    vals = []
                    for u in range(_U):
                        cand = rs[u] + step
                        vals.append((cand, plsc.load_gather(vrow, [cand - 1])))
                    for u in range(_U):
                        cand, val = vals[u]
                        rs[u] = jnp.where(val < xs[u], cand, rs[u])
                for u in range(_U):
                    x = xs[u]
                    r = rs[u]
                    l = jnp.maximum(r - 1, 0)
                    rr = jnp.minimum(r, _KP - 1)
                    vl = plsc.load_gather(vrow, [l])
                    vr = plsc.load_gather(vrow, [rr])
                    ml_ = plsc.load_gather(mrow, [l])
                    mr_ = plsc.load_gather(mrow, [rr])
                    dl = x - vl
                    dr = x - vr
                    sql = dl * dl
                    sqr = dr * dr
                    big = jnp.float32(jnp.inf)
                    sql = jnp.where(r >= 1, sql, big)
                    sqr = jnp.where(r <= _K - 1, sqr, big)
                    mini = jnp.where(sql < sqr, ml_,
                                     jnp.where(sqr < sql, mr_, jnp.minimum(ml_, mr_)))
                    omin[pl.ds((qi * _U + u) * 16, 16)] = mini
                return carry

            jax.lax.fori_loop(0, _N // 16 // _U, qbody, 0)
            pltpu.sync_copy(omin, mi_hbm.at[j])

    return search_kernel


def _loss_kernel(f_ref, cpad_ref, fT_ref, extv_ref, exti_ref, miT_ref, out_ref):
    f = f_ref[...]

    fT = fT_ref[...]
    lo_vb = jnp.broadcast_to(extv_ref[:, 0:1], (_D, _N))
    hi_vb = jnp.broadcast_to(extv_ref[:, 1:2], (_D, _N))
    lo_ib = jnp.broadcast_to(exti_ref[:, 0:1], (_D, _N))
    hi_ib = jnp.broadcast_to(exti_ref[:, 1:2], (_D, _N))
    dl = fT - lo_vb
    dh = fT - hi_vb
    sql = dl * dl
    sqh = dh * dh
    maT = jnp.where(sql > sqh, lo_ib,
                    jnp.where(sqh > sql, hi_ib, jnp.minimum(lo_ib, hi_ib)))

    def mode(idxT):
        counts = jnp.zeros((_D, _N), jnp.int32)
        for r in range(_D):
            rolled = jnp.roll(idxT, r, axis=0) if r else idxT
            counts = counts + (rolled == idxT).astype(jnp.int32)
        key = counts * 1024 + (1023 - idxT)
        mkey = jnp.max(key, axis=0, keepdims=True)
        mode_row = 1023 - jnp.bitwise_and(mkey, 1023)
        return mode_row.reshape(_N, 1)

    mode_min = mode(miT_ref[...])
    mode_max = mode(maT)

    iota_m = jax.lax.broadcasted_iota(jnp.int32, (_N, _N), 1)
    oh_p = (mode_min == iota_m).astype(jnp.float32)
    oh_n = (mode_max == iota_m).astype(jnp.float32)
    cpad = cpad_ref[...]
    pos = jnp.dot(oh_p, cpad, preferred_element_type=jnp.float32)
    neg = jnp.dot(oh_n, cpad, preferred_element_type=jnp.float32)

    eps = jnp.float32(1e-6)

    def pdist(a, b):
        d = a - b + eps
        return jnp.sqrt(jnp.sum(d * d, axis=1, keepdims=True))

    d_ap = pdist(f, pos)
    d_an = pdist(f, neg)
    d_pn = pdist(pos, neg)
    d_neg = jnp.minimum(d_an, d_pn)
    li = jnp.maximum(d_ap - d_neg + 1.0, 0.0)
    loss = jnp.sum(li) * jnp.float32(1.0 / _N)
    out_ref[...] = jnp.full((8, 128), loss, jnp.float32)


def kernel(input_features, centroids):
    f = input_features.astype(jnp.float32)
    c = centroids.astype(jnp.float32)

    inf = jnp.float32(jnp.inf)
    cT = jnp.concatenate(
        [c, jnp.full((_KP - _K, _D), inf, jnp.float32)], axis=0).T
    fT = f.T

    vsort, msort = pl.pallas_call(
        _sort_kernel,
        out_shape=[jax.ShapeDtypeStruct((_D, _KP), jnp.float32),
                   jax.ShapeDtypeStruct((_D, _KP), jnp.int32)],
    )(cT)

    miT = _make_search_kernel()(vsort, msort, fT)

    zpadv = jnp.zeros((_D, 6), jnp.float32)
    zpadi = jnp.zeros((_D, 6), jnp.int32)
    extv = jnp.concatenate([vsort[:, 0:1], vsort[:, _K - 1:_K], zpadv], axis=1)
    exti = jnp.concatenate([msort[:, 0:1], msort[:, _K - 1:_K], zpadi], axis=1)

    cpad = jnp.concatenate([c, jnp.zeros((_N - _K, _D), jnp.float32)], axis=0)
    out = pl.pallas_call(
        _loss_kernel,
        out_shape=jax.ShapeDtypeStruct((8, 128), jnp.float32),
    )(f, cpad, fT, extv, exti, miT)
    return out[0, 0]


# speedup vs baseline: 1.2913x; 1.0393x over previous
import functools

import jax
import jax.numpy as jnp
from jax import lax
from jax.experimental import pallas as pl
from jax.experimental.pallas import tpu as pltpu
from jax.experimental.pallas import tpu_sc as plsc

_N = 1024
_D = 64
_K = 1000
_KP = 1024


def _sort_kernel(ct_ref, vs_ref, ms_ref):
    v = ct_ref[...]
    i = jax.lax.broadcasted_iota(jnp.int32, (_D, _KP), 1)
    lane = jax.lax.broadcasted_iota(jnp.int32, (_D, _KP), 1)

    for kk in range(1, 11):
        blk = 1 << kk
        desc = (lane & blk) != 0
        for jj in range(kk - 1, -1, -1):
            s = 1 << jj
            upper = (lane & s) != 0
            flip = jnp.logical_xor(upper, desc)
            pv = jnp.where(upper, jnp.roll(v, s, axis=1), jnp.roll(v, -s, axis=1))
            pi = jnp.where(upper, jnp.roll(i, s, axis=1), jnp.roll(i, -s, axis=1))
            lt = (pv < v) | ((pv == v) & upper)
            take = jnp.logical_xor(lt, flip)
            v = jnp.where(take, pv, v)
            i = jnp.where(take, pi, i)

    m = i
    for dd in [1, 2, 4, 8, 16, 32, 64, 128, 256, 512]:
        ok = (lane >= dd) & (v == jnp.roll(v, dd, axis=1))
        m = jnp.where(ok, jnp.minimum(m, jnp.roll(m, dd, axis=1)), m)

    vs_ref[...] = v
    ms_ref[...] = m


def _make_search_kernel():
    mesh = plsc.VectorSubcoreMesh(core_axis_name="c", subcore_axis_name="s")

    @functools.partial(
        pl.kernel,
        mesh=mesh,
        out_type=jax.ShapeDtypeStruct((_D, _KP), jnp.int32),
        compiler_params=pltpu.CompilerParams(needs_layout_passes=False),
        scratch_types=[
            pltpu.VMEM((_KP,), jnp.float32),
            pltpu.VMEM((_KP,), jnp.int32),
            pltpu.VMEM((_KP,), jnp.float32),
            pltpu.VMEM((_KP,), jnp.int32),
        ],
    )
    def search_kernel(vs_hbm, ms_hbm, ft_hbm, mi_hbm, vrow, mrow, frow, omin):
        wid = lax.axis_index("s") * 2 + lax.axis_index("c")
        for rj in range(2):
            j = wid * 2 + rj
            pltpu.sync_copy(vs_hbm.at[j], vrow)
            pltpu.sync_copy(ms_hbm.at[j], mrow)
            pltpu.sync_copy(ft_hbm.at[j], frow)

            _U = 8

            def qbody(qi, carry):
                xs = [frow[pl.ds((qi * _U + u) * 16, 16)] for u in range(_U)]
                rs = [jnp.zeros((16,), jnp.int32) for _ in range(_U)]
                for step in (512, 256, 128, 64, 32, 16, 8, 4, 2, 1):
                    vals = []
                    for u in range(_U):
                        cand = rs[u] + step
                        vals.append((cand, plsc.load_gather(vrow, [cand - 1])))
                    for u in range(_U):
                        cand, val = vals[u]
                        rs[u] = jnp.where(val < xs[u], cand, rs[u])
                for u in range(_U):
                    x = xs[u]
                    r = rs[u]
                    l = jnp.maximum(r - 1, 0)
                    rr = jnp.minimum(r, _KP - 1)
                    vl = plsc.load_gather(vrow, [l])
                    vr = plsc.load_gather(vrow, [rr])
                    ml_ = plsc.load_gather(mrow, [l])
                    mr_ = plsc.load_gather(mrow, [rr])
                    dl = x - vl
                    dr = x - vr
                    sql = dl * dl
                    sqr = dr * dr
                    big = jnp.float32(jnp.inf)
                    sql = jnp.where(r >= 1, sql, big)
                    sqr = jnp.where(r <= _K - 1, sqr, big)
                    mini = jnp.where(sql < sqr, ml_,
                                     jnp.where(sqr < sql, mr_, jnp.minimum(ml_, mr_)))
                    omin[pl.ds((qi * _U + u) * 16, 16)] = mini
                return carry

            jax.lax.fori_loop(0, _N // 16 // _U, qbody, 0)
            pltpu.sync_copy(omin, mi_hbm.at[j])

    return search_kernel


def _mode(idxT):
    counts = jnp.zeros((_D, _N), jnp.int32)
    for r in range(_D):
        rolled = jnp.roll(idxT, r, axis=0) if r else idxT
        counts = counts + (rolled == idxT).astype(jnp.int32)
    key = counts * 1024 + (1023 - idxT)
    mkey = jnp.max(key, axis=0, keepdims=True)
    mode_row = 1023 - jnp.bitwise_and(mkey, 1023)
    return mode_row.reshape(_N, 1)


def _onehot_rows(mode_col, cpad):
    iota_m = jax.lax.broadcasted_iota(jnp.int32, (_N, _N), 1)
    oh = (mode_col == iota_m).astype(jnp.float32)
    return jnp.dot(oh, cpad, preferred_element_type=jnp.float32)


def _negside_kernel(cpad_ref, fT_ref, extv_ref, exti_ref, neg_ref):
    fT = fT_ref[...]
    lo_vb = jnp.broadcast_to(extv_ref[:, 0:1], (_D, _N))
    hi_vb = jnp.broadcast_to(extv_ref[:, 1:2], (_D, _N))
    lo_ib = jnp.broadcast_to(exti_ref[:, 0:1], (_D, _N))
    hi_ib = jnp.broadcast_to(exti_ref[:, 1:2], (_D, _N))
    dl = fT - lo_vb
    dh = fT - hi_vb
    sql = dl * dl
    sqh = dh * dh
    maT = jnp.where(sql > sqh, lo_ib,
                    jnp.where(sqh > sql, hi_ib, jnp.minimum(lo_ib, hi_ib)))
    neg_ref[...] = _onehot_rows(_mode(maT), cpad_ref[...])


def _final_kernel(f_ref, cpad_ref, miT_ref, neg_ref, out_ref):
    f = f_ref[...]
    pos = _onehot_rows(_mode(miT_ref[...]), cpad_ref[...])
    neg = neg_ref[...]

    eps = jnp.float32(1e-6)

    def pdist(a, b):
        d = a - b + eps
        return jnp.sqrt(jnp.sum(d * d, axis=1, keepdims=True))

    d_ap = pdist(f, pos)
    d_an = pdist(f, neg)
    d_pn = pdist(pos, neg)
    d_neg = jnp.minimum(d_an, d_pn)
    li = jnp.maximum(d_ap - d_neg + 1.0, 0.0)
    loss = jnp.sum(li) * jnp.float32(1.0 / _N)
    out_ref[...] = jnp.full((8, 128), loss, jnp.float32)


def kernel(input_features, centroids):
    f = input_features.astype(jnp.float32)
    c = centroids.astype(jnp.float32)

    inf = jnp.float32(jnp.inf)
    cT = jnp.concatenate(
        [c, jnp.full((_KP - _K, _D), inf, jnp.float32)], axis=0).T
    fT = f.T

    vsort, msort = pl.pallas_call(
        _sort_kernel,
        out_shape=[jax.ShapeDtypeStruct((_D, _KP), jnp.float32),
                   jax.ShapeDtypeStruct((_D, _KP), jnp.int32)],
    )(cT)

    miT = _make_search_kernel()(vsort, msort, fT)

    zpadv = jnp.zeros((_D, 6), jnp.float32)
    zpadi = jnp.zeros((_D, 6), jnp.int32)
    extv = jnp.concatenate([vsort[:, 0:1], vsort[:, _K - 1:_K], zpadv], axis=1)
    exti = jnp.concatenate([msort[:, 0:1], msort[:, _K - 1:_K], zpadi], axis=1)

    cpad = jnp.concatenate([c, jnp.zeros((_N - _K, _D), jnp.float32)], axis=0)
    neg = pl.pallas_call(
        _negside_kernel,
        out_shape=jax.ShapeDtypeStruct((_N, _D), jnp.float32),
    )(cpad, fT, extv, exti)
    out = pl.pallas_call(
        _final_kernel,
        out_shape=jax.ShapeDtypeStruct((8, 128), jnp.float32),
    )(f, cpad, miT, neg)
    return out[0, 0]


# speedup vs baseline: 1.3430x; 1.0400x over previous
import functools

import jax
import jax.numpy as jnp
from jax import lax
from jax.experimental import pallas as pl
from jax.experimental.pallas import tpu as pltpu
from jax.experimental.pallas import tpu_sc as plsc

_N = 1024
_D = 64
_K = 1000
_KP = 1024


def _sort_kernel(ct_ref, vs_ref, ms_ref):
    v = ct_ref[...]
    i = jax.lax.broadcasted_iota(jnp.int32, (_D, _KP), 1)
    lane = jax.lax.broadcasted_iota(jnp.int32, (_D, _KP), 1)

    for kk in range(1, 11):
        blk = 1 << kk
        desc = (lane & blk) != 0
        for jj in range(kk - 1, -1, -1):
            s = 1 << jj
            upper = (lane & s) != 0
            flip = jnp.logical_xor(upper, desc)
            pv = jnp.where(upper, jnp.roll(v, s, axis=1), jnp.roll(v, -s, axis=1))
            pi = jnp.where(upper, jnp.roll(i, s, axis=1), jnp.roll(i, -s, axis=1))
            lt = (pv < v) | ((pv == v) & upper)
            take = jnp.logical_xor(lt, flip)
            v = jnp.where(take, pv, v)
            i = jnp.where(take, pi, i)

    m = i
    for dd in [1, 2, 4, 8, 16, 32, 64, 128, 256, 512]:
        ok = (lane >= dd) & (v == jnp.roll(v, dd, axis=1))
        m = jnp.where(ok, jnp.minimum(m, jnp.roll(m, dd, axis=1)), m)

    vs_ref[...] = v
    ms_ref[...] = m


def _make_search_kernel():
    mesh = plsc.VectorSubcoreMesh(core_axis_name="c", subcore_axis_name="s")

    @functools.partial(
        pl.kernel,
        mesh=mesh,
        out_type=jax.ShapeDtypeStruct((_D, _KP), jnp.int32),
        compiler_params=pltpu.CompilerParams(needs_layout_passes=False),
        scratch_types=[
            pltpu.VMEM((_KP,), jnp.float32),
            pltpu.VMEM((_KP,), jnp.int32),
            pltpu.VMEM((_KP,), jnp.float32),
            pltpu.VMEM((_KP,), jnp.int32),
        ],
    )
    def search_kernel(vs_hbm, ms_hbm, ft_hbm, mi_hbm, vrow, mrow, frow, omin):
        wid = lax.axis_index("s") * 2 + lax.axis_index("c")
        for rj in range(2):
            j = wid * 2 + rj
            pltpu.sync_copy(vs_hbm.at[j], vrow)
            pltpu.sync_copy(ms_hbm.at[j], mrow)
            pltpu.sync_copy(ft_hbm.at[j], frow)

            _U = 8

            def qbody(qi, carry):
                bidx = lax.iota(jnp.int32, 16) * 64 + 63
                bnds = plsc.load_gather(vrow, [bidx])
                xs = [frow[pl.ds((qi * _U + u) * 16, 16)] for u in range(_U)]
                rs = [jnp.zeros((16,), jnp.int32) for _ in range(_U)]
                for step in (512, 256, 128, 64):
                    for u in range(_U):
                        cand = rs[u] + step
                        val = jnp.take_along_axis(
                            bnds, (cand >> 6) - 1, axis=0,
                            mode="promise_in_bounds")
                        rs[u] = jnp.where(val < xs[u], cand, rs[u])
                for step in (32, 16, 8, 4, 2, 1):
                    vals = []
                    for u in range(_U):
                        cand = rs[u] + step
                        vals.append((cand, plsc.load_gather(vrow, [cand - 1])))
                    for u in range(_U):
                        cand, val = vals[u]
                        rs[u] = jnp.where(val < xs[u], cand, rs[u])
                for u in range(_U):
                    x = xs[u]
                    r = rs[u]
                    l = jnp.maximum(r - 1, 0)
                    rr = jnp.minimum(r, _KP - 1)
                    vl = plsc.load_gather(vrow, [l])
                    vr = plsc.load_gather(vrow, [rr])
                    ml_ = plsc.load_gather(mrow, [l])
                    mr_ = plsc.load_gather(mrow, [rr])
                    dl = x - vl
                    dr = x - vr
                    sql = dl * dl
                    sqr = dr * dr
                    big = jnp.float32(jnp.inf)
                    sql = jnp.where(r >= 1, sql, big)
                    sqr = jnp.where(r <= _K - 1, sqr, big)
                    mini = jnp.where(sql < sqr, ml_,
                                     jnp.where(sqr < sql, mr_, jnp.minimum(ml_, mr_)))
                    omin[pl.ds((qi * _U + u) * 16, 16)] = mini
                return carry

            jax.lax.fori_loop(0, _N // 16 // _U, qbody, 0)
            pltpu.sync_copy(omin, mi_hbm.at[j])

    return search_kernel


def _mode(idxT):
    counts = jnp.zeros((_D, _N), jnp.int32)
    for r in range(_D):
        rolled = jnp.roll(idxT, r, axis=0) if r else idxT
        counts = counts + (rolled == idxT).astype(jnp.int32)
    key = counts * 1024 + (1023 - idxT)
    mkey = jnp.max(key, axis=0, keepdims=True)
    mode_row = 1023 - jnp.bitwise_and(mkey, 1023)
    return mode_row.reshape(_N, 1)


def _onehot_rows(mode_col, cpad):
    iota_m = jax.lax.broadcasted_iota(jnp.int32, (_N, _N), 1)
    oh = (mode_col == iota_m).astype(jnp.float32)
    return jnp.dot(oh, cpad, preferred_element_type=jnp.float32)


def _negside_kernel(cpad_ref, fT_ref, extv_ref, exti_ref, neg_ref):
    fT = fT_ref[...]
    lo_vb = jnp.broadcast_to(extv_ref[:, 0:1], (_D, _N))
    hi_vb = jnp.broadcast_to(extv_ref[:, 1:2], (_D, _N))
    lo_ib = jnp.broadcast_to(exti_ref[:, 0:1], (_D, _N))
    hi_ib = jnp.broadcast_to(exti_ref[:, 1:2], (_D, _N))
    dl = fT - lo_vb
    dh = fT - hi_vb
    sql = dl * dl
    sqh = dh * dh
    maT = jnp.where(sql > sqh, lo_ib,
                    jnp.where(sqh > sql, hi_ib, jnp.minimum(lo_ib, hi_ib)))
    neg_ref[...] = _onehot_rows(_mode(maT), cpad_ref[...])


def _final_kernel(f_ref, cpad_ref, miT_ref, neg_ref, out_ref):
    f = f_ref[...]
    pos = _onehot_rows(_mode(miT_ref[...]), cpad_ref[...])
    neg = neg_ref[...]

    eps = jnp.float32(1e-6)

    def pdist(a, b):
        d = a - b + eps
        return jnp.sqrt(jnp.sum(d * d, axis=1, keepdims=True))

    d_ap = pdist(f, pos)
    d_an = pdist(f, neg)
    d_pn = pdist(pos, neg)
    d_neg = jnp.minimum(d_an, d_pn)
    li = jnp.maximum(d_ap - d_neg + 1.0, 0.0)
    loss = jnp.sum(li) * jnp.float32(1.0 / _N)
    out_ref[...] = jnp.full((8, 128), loss, jnp.float32)


def kernel(input_features, centroids):
    f = input_features.astype(jnp.float32)
    c = centroids.astype(jnp.float32)

    inf = jnp.float32(jnp.inf)
    cT = jnp.concatenate(
        [c, jnp.full((_KP - _K, _D), inf, jnp.float32)], axis=0).T
    fT = f.T

    vsort, msort = pl.pallas_call(
        _sort_kernel,
        out_shape=[jax.ShapeDtypeStruct((_D, _KP), jnp.float32),
                   jax.ShapeDtypeStruct((_D, _KP), jnp.int32)],
    )(cT)

    miT = _make_search_kernel()(vsort, msort, fT)

    zpadv = jnp.zeros((_D, 6), jnp.float32)
    zpadi = jnp.zeros((_D, 6), jnp.int32)
    extv = jnp.concatenate([vsort[:, 0:1], vsort[:, _K - 1:_K], zpadv], axis=1)
    exti = jnp.concatenate([msort[:, 0:1], msort[:, _K - 1:_K], zpadi], axis=1)

    cpad = jnp.concatenate([c, jnp.zeros((_N - _K, _D), jnp.float32)], axis=0)
    neg = pl.pallas_call(
        _negside_kernel,
        out_shape=jax.ShapeDtypeStruct((_N, _D), jnp.float32),
    )(cpad, fT, extv, exti)
    out = pl.pallas_call(
        _final_kernel,
        out_shape=jax.ShapeDtypeStruct((8, 128), jnp.float32),
    )(f, cpad, miT, neg)
    return out[0, 0]
